# baseline (speedup 1.0000x reference)
import functools

import jax
import jax.numpy as jnp
from jax import lax
from jax.experimental import pallas as pl
from jax.experimental.pallas import tpu as pltpu
from jax.experimental.pallas import tpu_sc as plsc

NC = 2
NS = 16
LANES = 16

B = 64
C = 384
W = 1024

NW = NC * NS
BPW = B // NW
CH = 32
NCHUNK = C // CH


def _sc_body(x_hbm, label_hbm, bias_hbm, out_hbm,
             label_v, bias_v, buf0, buf1,
             sin0, sin1, sout0, sout1):
    wid = lax.axis_index("s") * NC + lax.axis_index("c")

    pltpu.sync_copy(label_hbm, label_v)
    pltpu.sync_copy(bias_hbm, bias_v)

    bufs = (buf0, buf1)
    sins = (sin0, sin1)
    souts = (sout0, sout1)

    def add_bias_chunk(buf, lab_idx, c0):
        def row_body(r, _):
            cvec = jnp.full((LANES,), c0 + r, jnp.int32)
            bvec = plsc.load_gather(bias_v, [lab_idx, cvec])

            def col_body(j, _):
                sl = pl.ds(pl.multiple_of(j * LANES, LANES), LANES)
                plsc.addupdate(buf.at[r, sl], bvec)
                return 0

            lax.fori_loop(0, W // LANES, col_body, 0)
            return 0

        lax.fori_loop(0, CH, row_body, 0)

    for bi in range(BPW):
        b = wid * BPW + bi
        lab_idx = label_v[b, pl.ds(0, LANES)]

        in_cp = [None] * NCHUNK
        out_cp = [None] * NCHUNK
        in_cp[0] = pltpu.make_async_copy(
            x_hbm.at[b, pl.ds(0, CH), :], bufs[0], sins[0])
        in_cp[0].start()
        for k in range(NCHUNK):
            cur = k % 2
            nxt = 1 - cur
            if k + 1 < NCHUNK:
                if k >= 1:
                    out_cp[k - 1].wait()
                in_cp[k + 1] = pltpu.make_async_copy(
                    x_hbm.at[b, pl.ds((k + 1) * CH, CH), :], bufs[nxt], sins[nxt])
                in_cp[k + 1].start()
            in_cp[k].wait()
            add_bias_chunk(bufs[cur], lab_idx, k * CH)
            out_cp[k] = pltpu.make_async_copy(
                bufs[cur], out_hbm.at[b, pl.ds(k * CH, CH), :], souts[cur])
            out_cp[k].start()
        out_cp[NCHUNK - 2].wait()
        out_cp[NCHUNK - 1].wait()


@jax.jit
def _label_norm_sc(x3, label, bias_table):
    mesh = plsc.VectorSubcoreMesh(
        core_axis_name="c", subcore_axis_name="s",
        num_cores=NC, num_subcores=NS)
    return pl.kernel(
        _sc_body,
        out_type=jax.ShapeDtypeStruct((B, C, W), jnp.float32),
        mesh=mesh,
        scratch_types=[
            pltpu.VMEM((B, LANES), jnp.int32),
            pltpu.VMEM((2, C), jnp.float32),
            pltpu.VMEM((CH, W), jnp.float32),
            pltpu.VMEM((CH, W), jnp.float32),
            pltpu.SemaphoreType.DMA,
            pltpu.SemaphoreType.DMA,
            pltpu.SemaphoreType.DMA,
            pltpu.SemaphoreType.DMA,
        ],
        compiler_params=pltpu.CompilerParams(needs_layout_passes=False),
    )(x3, label, bias_table)


def kernel(x, label, bias_table):
    x3 = x.reshape(B, C, W)
    lab_b = jnp.broadcast_to(label.astype(jnp.int32)[:, None], (B, LANES))
    out = _label_norm_sc(x3, lab_b, bias_table)
    return out.reshape(x.shape)


# speedup vs baseline: 1.5959x; 1.5959x over previous
import functools

import jax
import jax.numpy as jnp
from jax import lax
from jax.experimental import pallas as pl
from jax.experimental.pallas import tpu as pltpu
from jax.experimental.pallas import tpu_sc as plsc

NC = 2
NS = 16
LANES = 16

B = 64
C = 384
W = 1024

NW = NC * NS
BPW = B // NW
CH = 32
NCHUNK = C // CH


def _sc_body(x_hbm, label_hbm, bias_hbm, out_hbm,
             label_v, bias_v, buf0, buf1,
             sin0, sin1, sout0, sout1):
    wid = lax.axis_index("s") * NC + lax.axis_index("c")

    pltpu.sync_copy(label_hbm, label_v)
    pltpu.sync_copy(bias_hbm, bias_v)

    bufs = (buf0, buf1)
    sins = (sin0, sin1)
    souts = (sout0, sout1)

    def add_bias_chunk(buf, lab_idx, c0):
        @plsc.parallel_loop(0, CH)
        def row_body(r):
            cvec = jnp.full((LANES,), c0 + r, jnp.int32)
            bvec = plsc.load_gather(bias_v, [lab_idx, cvec])

            @plsc.parallel_loop(0, W // LANES, unroll=8)
            def col_body(j):
                sl = pl.ds(pl.multiple_of(j * LANES, LANES), LANES)
                plsc.addupdate(buf.at[r, sl], bvec)

    for bi in range(BPW):
        b = wid * BPW + bi
        lab_idx = label_v[b, pl.ds(0, LANES)]

        in_cp = [None] * NCHUNK
        out_cp = [None] * NCHUNK
        in_cp[0] = pltpu.make_async_copy(
            x_hbm.at[b, pl.ds(0, CH), :], bufs[0], sins[0])
        in_cp[0].start()
        for k in range(NCHUNK):
            cur = k % 2
            nxt = 1 - cur
            if k + 1 < NCHUNK:
                if k >= 1:
                    out_cp[k - 1].wait()
                in_cp[k + 1] = pltpu.make_async_copy(
                    x_hbm.at[b, pl.ds((k + 1) * CH, CH), :], bufs[nxt], sins[nxt])
                in_cp[k + 1].start()
            in_cp[k].wait()
            add_bias_chunk(bufs[cur], lab_idx, k * CH)
            out_cp[k] = pltpu.make_async_copy(
                bufs[cur], out_hbm.at[b, pl.ds(k * CH, CH), :], souts[cur])
            out_cp[k].start()
        out_cp[NCHUNK - 2].wait()
        out_cp[NCHUNK - 1].wait()


@jax.jit
def _label_norm_sc(x3, label, bias_table):
    mesh = plsc.VectorSubcoreMesh(
        core_axis_name="c", subcore_axis_name="s",
        num_cores=NC, num_subcores=NS)
    return pl.kernel(
        _sc_body,
        out_type=jax.ShapeDtypeStruct((B, C, W), jnp.float32),
        mesh=mesh,
        scratch_types=[
            pltpu.VMEM((B, LANES), jnp.int32),
            pltpu.VMEM((2, C), jnp.float32),
            pltpu.VMEM((CH, W), jnp.float32),
            pltpu.VMEM((CH, W), jnp.float32),
            pltpu.SemaphoreType.DMA,
            pltpu.SemaphoreType.DMA,
            pltpu.SemaphoreType.DMA,
            pltpu.SemaphoreType.DMA,
        ],
        compiler_params=pltpu.CompilerParams(needs_layout_passes=False),
    )(x3, label, bias_table)


def kernel(x, label, bias_table):
    x3 = x.reshape(B, C, W)
    lab_b = jnp.broadcast_to(label.astype(jnp.int32)[:, None], (B, LANES))
    out = _label_norm_sc(x3, lab_b, bias_table)
    return out.reshape(x.shape)


# speedup vs baseline: 4.3255x; 2.7104x over previous
import jax
import jax.numpy as jnp
from jax import lax
from jax.experimental import pallas as pl
from jax.experimental.pallas import tpu as pltpu
from jax.experimental.pallas import tpu_sc as plsc

NC = 2
NS = 16
LANES = 16

B = 64
C = 384
HW = 1024
CT = C // LANES

NW = NC * NS
BPW = B // NW
CHW = 64
NCHUNK = HW // CHW


def _sc_body(x_hbm, label_hbm, bias_hbm, out_hbm,
             label_v, bias_v, buf0, buf1,
             sin0, sin1, sout0, sout1):
    wid = lax.axis_index("s") * NC + lax.axis_index("c")

    pltpu.sync_copy(label_hbm, label_v)
    pltpu.sync_copy(bias_hbm, bias_v)

    bufs = (buf0, buf1)
    sins = (sin0, sin1)
    souts = (sout0, sout1)

    for bi in range(BPW):
        b = wid * BPW + bi
        lab_vec = label_v[b, pl.ds(0, LANES)]
        sel = lab_vec >= 1
        brow = [
            jnp.where(sel,
                      bias_v[1, pl.ds(t * LANES, LANES)],
                      bias_v[0, pl.ds(t * LANES, LANES)])
            for t in range(CT)
        ]

        in_cp = [None] * NCHUNK
        out_cp = [None] * NCHUNK
        in_cp[0] = pltpu.make_async_copy(
            x_hbm.at[b, pl.ds(0, CHW), :], bufs[0], sins[0])
        in_cp[0].start()
        for k in range(NCHUNK):
            cur = k % 2
            nxt = 1 - cur
            if k + 1 < NCHUNK:
                if k >= 1:
                    out_cp[k - 1].wait()
                in_cp[k + 1] = pltpu.make_async_copy(
                    x_hbm.at[b, pl.ds((k + 1) * CHW, CHW), :], bufs[nxt], sins[nxt])
                in_cp[k + 1].start()
            in_cp[k].wait()

            buf = bufs[cur]

            @plsc.parallel_loop(0, CHW)
            def row_body(r):
                for t in range(CT):
                    sl = pl.ds(t * LANES, LANES)
                    plsc.addupdate(buf.at[r, sl], brow[t])

            out_cp[k] = pltpu.make_async_copy(
                buf, out_hbm.at[b, pl.ds(k * CHW, CHW), :], souts[cur])
            out_cp[k].start()
        out_cp[NCHUNK - 2].wait()
        out_cp[NCHUNK - 1].wait()


@jax.jit
def _label_norm_sc(xt, label_b, bias_table):
    mesh = plsc.VectorSubcoreMesh(
        core_axis_name="c", subcore_axis_name="s",
        num_cores=NC, num_subcores=NS)
    return pl.kernel(
        _sc_body,
        out_type=jax.ShapeDtypeStruct((B, HW, C), jnp.float32),
        mesh=mesh,
        scratch_types=[
            pltpu.VMEM((B, LANES), jnp.int32),
            pltpu.VMEM((2, C), jnp.float32),
            pltpu.VMEM((CHW, C), jnp.float32),
            pltpu.VMEM((CHW, C), jnp.float32),
            pltpu.SemaphoreType.DMA,
            pltpu.SemaphoreType.DMA,
            pltpu.SemaphoreType.DMA,
            pltpu.SemaphoreType.DMA,
        ],
        compiler_params=pltpu.CompilerParams(needs_layout_passes=False),
    )(xt, label_b, bias_table)


def kernel(x, label, bias_table):
    xt = jnp.transpose(x.reshape(B, C, HW), (0, 2, 1))
    lab_b = jnp.broadcast_to(label.astype(jnp.int32)[:, None], (B, LANES))
    out = _label_norm_sc(xt, lab_b, bias_table)
    return jnp.transpose(out, (0, 2, 1)).reshape(x.shape)
